# baseline (speedup 1.0000x reference)
import dataclasses
import functools

import jax
import jax.numpy as jnp
from jax import lax
from jax.experimental import pallas as pl
from jax.experimental.pallas import tpu as pltpu
from jax.experimental.pallas import tpu_sc as plsc

NUM_CORES = 2
NUM_SUBCORES = 16
NUM_WORKERS = NUM_CORES * NUM_SUBCORES
LANES = 16
NBUF = 4
ROW = 128


def kernel(token_ids, weights):
    B, S = token_ids.shape
    V, D = weights.shape
    chunk = B // NUM_WORKERS
    assert B % NUM_WORKERS == 0 and chunk == 128 and D == 64
    idx_t = token_ids.T.reshape(S, NUM_WORKERS, chunk).transpose(1, 0, 2)
    table_p = jnp.pad(weights, ((0, 0), (0, ROW - D)))

    mesh = plsc.VectorSubcoreMesh(core_axis_name="c", subcore_axis_name="s")
    cp = pltpu.CompilerParams()
    if "needs_layout_passes" in pltpu.CompilerParams.__dataclass_fields__:
        cp = dataclasses.replace(cp, needs_layout_passes=False)

    @functools.partial(
        pl.kernel,
        mesh=mesh,
        compiler_params=cp,
        out_type=jax.ShapeDtypeStruct((S, D, B), jnp.float32),
        scratch_types=[
            pltpu.VMEM((S, chunk), jnp.int32),
            pltpu.VMEM((NBUF, chunk, ROW), jnp.float32),
            pltpu.VMEM((2, D, chunk), jnp.float32),
            pltpu.SemaphoreType.DMA((NBUF,)),
            pltpu.SemaphoreType.DMA((2,)),
        ],
    )
    def gather_kernel(table_hbm, idx_hbm, out_hbm, idx_v, rows_v, out_v,
                      gsem, osem):
        wid = lax.axis_index("s") * NUM_CORES + lax.axis_index("c")
        b0 = wid * chunk
        iota = lax.iota(jnp.int32, LANES)

        def start_gather(j, b):
            pltpu.async_copy(table_hbm.at[idx_v.at[j]], rows_v.at[b],
                             gsem.at[b])

        pltpu.sync_copy(idx_hbm.at[wid], idx_v)
        for w in range(3):
            start_gather(w, w)

        @pl.loop(0, S, step=2)
        def _(j):
            for p in range(2):
                jj = j + p
                b = jj % NBUF
                pltpu.make_async_copy(table_hbm.at[idx_v.at[jj]],
                                      rows_v.at[b], gsem.at[b]).wait()
                @pl.when(jj + 3 < S)
                def _():
                    start_gather(jj + 3, (jj + 3) % NBUF)

                @pl.when(jj >= 2)
                def _():
                    pltpu.make_async_copy(
                        out_v.at[p], out_hbm.at[0, :, pl.ds(b0, chunk)],
                        osem.at[p]).wait()

                rows2 = rows_v.at[b]
                outp = out_v.at[p]
                @plsc.parallel_loop(0, D, unroll=4)
                def _(i):
                    dv = ((iota + i) & (LANES - 1)) + (i & ~(LANES - 1))
                    for tg in range(chunk // LANES):
                        tv = iota + (tg * LANES)
                        vals = plsc.load_gather(rows2, [tv, dv])
                        plsc.store_scatter(outp, [dv, tv], vals)

                pltpu.async_copy(out_v.at[p],
                                 out_hbm.at[jj, :, pl.ds(b0, chunk)],
                                 osem.at[p])

        for jj in (S - 2, S - 1):
            pltpu.make_async_copy(out_v.at[jj % 2],
                                  out_hbm.at[0, :, pl.ds(b0, chunk)],
                                  osem.at[jj % 2]).wait()

    out_t = gather_kernel(table_p, idx_t)
    return out_t.transpose(2, 0, 1)
